# baseline (speedup 1.0000x reference)
import functools

import jax
import jax.numpy as jnp
import numpy as np
from jax import lax
from jax.experimental import pallas as pl
from jax.experimental.pallas import tpu as pltpu
from jax.experimental.pallas import tpu_sc as plsc

N_NODES = 100000
EMBED_D = 64
BATCH = 1024
TILE_N = 1000
OFF_TILE = 2000
N_OFF_TILES = 15
N_OFF = OFF_TILE * N_OFF_TILES
COL_SPLIT = N_NODES - N_OFF
A_TILES = COL_SPLIT // TILE_N

_U32_9 = np.uint32(9)
_EXP_ONE = np.uint32(0x3F800000)
_MINVAL = np.float32(1e-10)
_SCALE = np.float32(np.float32(1.0) - np.float32(1e-10))
_LSE_SHIFT = np.float32(40.0)


def _threefry_bits(x1):
    ks = (np.uint32(0), np.uint32(1), np.uint32(0x1BD11BDB))
    rotations = ((13, 15, 26, 6), (17, 29, 16, 24))
    r = 13
    x0 = x1
    x1 = lax.shift_left(x1, np.uint32(r)) | lax.shift_right_logical(
        x1, np.uint32(32 - r))
    x1 = x1 ^ x0
    first = True
    for i in range(5):
        for r in rotations[i % 2]:
            if first:
                first = False
                continue
            x0 = x0 + x1
            x1 = lax.shift_left(x1, np.uint32(r)) | lax.shift_right_logical(
                x1, np.uint32(32 - r))
            x1 = x1 ^ x0
        x0 = x0 + ks[(i + 1) % 3]
        x1 = x1 + np.uint32(ks[(i + 2) % 3] + np.uint32(i + 1))
    return x0 ^ x1


def _bits_to_gumbel(bits):
    fbits = lax.shift_right_logical(bits, _U32_9) | _EXP_ONE
    u = lax.bitcast_convert_type(fbits, jnp.float32) - np.float32(1.0)
    u = jnp.maximum(_MINVAL, u * _SCALE + _MINVAL)
    return -jnp.log(-jnp.log(u))


def _online_update(j, tile_w, col0, scores, t, lane,
                   bv_ref, bi_ref, bs_ref, s_ref):
    tmax = jnp.max(t, axis=-1, keepdims=True)
    larg = jnp.min(jnp.where(t == tmax, lane, tile_w), axis=-1, keepdims=True)
    sel = jnp.sum(jnp.where(lane == larg, scores, 0.0), axis=-1, keepdims=True)
    upd = tmax > bv_ref[...]
    bv_ref[...] = jnp.where(upd, tmax, bv_ref[...])
    bi_ref[...] = jnp.where(upd, larg + (col0 + j * tile_w), bi_ref[...])
    bs_ref[...] = jnp.where(upd, sel, bs_ref[...])
    s_ref[...] += jnp.sum(jnp.exp(scores - _LSE_SHIFT), axis=-1, keepdims=True)


def _score_tile(ce_ref, emb_ref, bias_ref, tile_w):
    scores = lax.dot_general(
        ce_ref[...], emb_ref[...],
        dimension_numbers=(((1,), (1,)), ((), ())),
        preferred_element_type=jnp.float32)
    return scores + jnp.reshape(bias_ref[...], (1, tile_w))


def _a_body(ce_ref, emb_ref, bias_ref, bv_out, bi_out, bs_out, s_out,
            bv_ref, bi_ref, bs_ref, s_ref):
    j = pl.program_id(0)

    @pl.when(j == 0)
    def _init():
        bv_ref[...] = jnp.full((BATCH, 1), -jnp.inf, jnp.float32)
        bi_ref[...] = jnp.zeros((BATCH, 1), jnp.int32)
        bs_ref[...] = jnp.zeros((BATCH, 1), jnp.float32)
        s_ref[...] = jnp.zeros((BATCH, 1), jnp.float32)

    scores = _score_tile(ce_ref, emb_ref, bias_ref, TILE_N)
    lane = lax.broadcasted_iota(jnp.int32, (BATCH, TILE_N), 1)
    base = (lax.broadcasted_iota(jnp.int32, (BATCH, 1), 0) * N_NODES
            + (j * TILE_N + 1))
    bits = _threefry_bits((base + lane).astype(jnp.uint32))
    t = scores + _bits_to_gumbel(bits)
    _online_update(j, TILE_N, 0, scores, t, lane, bv_ref, bi_ref, bs_ref, s_ref)

    @pl.when(j == A_TILES - 1)
    def _finish():
        bv_out[...] = bv_ref[...]
        bi_out[...] = bi_ref[...]
        bs_out[...] = bs_ref[...]
        s_out[...] = s_ref[...]


def _b_body(ce_ref, emb_ref, bias_ref, bits_ref,
            bv_in, bi_in, bs_in, s_in, samp_ref, lp_ref,
            bv_ref, bi_ref, bs_ref, s_ref):
    j = pl.program_id(0)

    @pl.when(j == 0)
    def _init():
        bv_ref[...] = bv_in[...]
        bi_ref[...] = bi_in[...]
        bs_ref[...] = bs_in[...]
        s_ref[...] = s_in[...]

    scores = _score_tile(ce_ref, emb_ref, bias_ref, OFF_TILE)
    lane = lax.broadcasted_iota(jnp.int32, (BATCH, OFF_TILE), 1)
    t = scores + _bits_to_gumbel(jnp.reshape(bits_ref[...],
                                             (BATCH, OFF_TILE)))
    _online_update(j, OFF_TILE, COL_SPLIT, scores, t, lane,
                   bv_ref, bi_ref, bs_ref, s_ref)

    @pl.when(j == N_OFF_TILES - 1)
    def _finish():
        samp_ref[...] = bi_ref[...]
        lp_ref[...] = bs_ref[...] - (_LSE_SHIFT + jnp.log(s_ref[...]))


def _fused_tc(center_emb, embedding, bias, bits):
    state_shape = [jax.ShapeDtypeStruct((BATCH, 1), jnp.float32),
                   jax.ShapeDtypeStruct((BATCH, 1), jnp.int32),
                   jax.ShapeDtypeStruct((BATCH, 1), jnp.float32),
                   jax.ShapeDtypeStruct((BATCH, 1), jnp.float32)]
    state_specs = [pl.BlockSpec((BATCH, 1), lambda j: (0, 0))] * 4
    bias_a = bias[:COL_SPLIT].reshape(A_TILES, 1, TILE_N)
    bias_b = bias[COL_SPLIT:].reshape(N_OFF_TILES, 1, OFF_TILE)
    scratch = [pltpu.VMEM((BATCH, 1), jnp.float32),
               pltpu.VMEM((BATCH, 1), jnp.int32),
               pltpu.VMEM((BATCH, 1), jnp.float32),
               pltpu.VMEM((BATCH, 1), jnp.float32)]

    state = pl.pallas_call(
        _a_body,
        grid=(A_TILES,),
        in_specs=[
            pl.BlockSpec((BATCH, EMBED_D), lambda j: (0, 0)),
            pl.BlockSpec((TILE_N, EMBED_D), lambda j: (j, 0)),
            pl.BlockSpec((1, 1, TILE_N), lambda j: (j, 0, 0)),
        ],
        out_specs=state_specs,
        out_shape=state_shape,
        scratch_shapes=scratch,
        compiler_params=pltpu.CompilerParams(
            dimension_semantics=("arbitrary",)),
    )(center_emb, embedding, bias_a)

    samples2d, lp2d = pl.pallas_call(
        _b_body,
        grid=(N_OFF_TILES,),
        in_specs=[
            pl.BlockSpec((BATCH, EMBED_D), lambda j: (0, 0)),
            pl.BlockSpec((OFF_TILE, EMBED_D),
                         lambda j: (COL_SPLIT // OFF_TILE + j, 0)),
            pl.BlockSpec((1, 1, OFF_TILE), lambda j: (j, 0, 0)),
            pl.BlockSpec((1, BATCH, OFF_TILE), lambda j: (j, 0, 0)),
            *state_specs,
        ],
        out_specs=[
            pl.BlockSpec((BATCH, 1), lambda j: (0, 0)),
            pl.BlockSpec((BATCH, 1), lambda j: (0, 0)),
        ],
        out_shape=[
            jax.ShapeDtypeStruct((BATCH, 1), jnp.int32),
            jax.ShapeDtypeStruct((BATCH, 1), jnp.float32),
        ],
        scratch_shapes=scratch,
        compiler_params=pltpu.CompilerParams(
            dimension_semantics=("arbitrary",)),
    )(center_emb, embedding, bias_b, bits, *state)
    return samples2d[:, 0], lp2d[:, 0]


def _sc_bits():
    info = plsc.get_sparse_core_info()
    nc, ns = info.num_cores, info.num_subcores
    ntec = nc * ns
    rows_per_tec = BATCH // ntec
    mesh = plsc.VectorSubcoreMesh(core_axis_name="c", subcore_axis_name="s")

    @functools.partial(
        pl.kernel, mesh=mesh,
        out_type=jax.ShapeDtypeStruct((N_OFF_TILES, BATCH, OFF_TILE),
                                      jnp.uint32),
        scratch_types=[
            pltpu.VMEM((4, OFF_TILE), jnp.uint32),
            pltpu.SemaphoreType.DMA,
        ],
    )
    def bits_kernel(out_hbm, buf, sem):
        wid = lax.axis_index("s") * nc + lax.axis_index("c")
        r0 = wid * rows_per_tec
        iota16 = lax.iota(jnp.int32, 16)
        unroll = 5
        n_units = rows_per_tec * N_OFF_TILES

        def drain_one():
            pltpu.make_async_copy(out_hbm.at[0, 0], buf.at[0], sem).wait()

        def quad_body(q, _):
            @pl.when(q > 0)
            def _():
                for _i in range(4):
                    drain_one()
            for h in range(4):
                rk = q * 4 + h
                r = rk // N_OFF_TILES
                k = rk % N_OFF_TILES
                base = (r0 + r) * N_NODES + (COL_SPLIT + k * OFF_TILE + 1)

                def chunk(c, _, h=h, base=base):
                    off = c * (16 * unroll)
                    for v in range(unroll):
                        x1 = (base + (off + v * 16)
                              + iota16).astype(jnp.uint32)
                        buf[h, pl.ds(off + v * 16, 16)] = _threefry_bits(x1)
                    return ()

                lax.fori_loop(0, OFF_TILE // (16 * unroll), chunk, ())
                pltpu.async_copy(buf.at[h], out_hbm.at[k, r0 + r], sem)
            return ()

        lax.fori_loop(0, n_units // 4, quad_body, ())
        for _i in range(4):
            drain_one()

    return bits_kernel()


def _gather_sc(center_ids, embedding):
    info = plsc.get_sparse_core_info()
    nc = info.num_cores
    b_per_c = BATCH // nc
    mesh = plsc.ScalarSubcoreMesh(axis_name="c", num_cores=nc)

    @functools.partial(
        pl.kernel, mesh=mesh,
        out_type=jax.ShapeDtypeStruct((BATCH, EMBED_D), jnp.float32),
        scratch_types=[
            pltpu.SMEM((b_per_c,), jnp.int32),
            pltpu.SemaphoreType.DMA,
            pltpu.SemaphoreType.DMA,
        ],
    )
    def gather(table_hbm, idx_hbm, out_hbm, idx_s, isem, sem):
        base = lax.axis_index("c") * b_per_c
        pltpu.async_copy(idx_hbm.at[pl.ds(base, b_per_c)], idx_s, isem).wait()

        def body(i, _):
            pltpu.async_copy(table_hbm.at[pl.ds(idx_s[i], 1), :],
                             out_hbm.at[pl.ds(base + i, 1), :], sem)
            return ()

        lax.fori_loop(0, b_per_c, body, (), unroll=8)
        pltpu.make_async_copy(
            table_hbm.at[pl.ds(0, b_per_c), :],
            out_hbm.at[pl.ds(base, b_per_c), :], sem).wait()

    return gather(embedding, center_ids)


def kernel(center_ids, embedding, bias):
    center_emb = _gather_sc(center_ids.astype(jnp.int32), embedding)
    bits = _sc_bits()
    return _fused_tc(center_emb, embedding, bias, bits)


# speedup vs baseline: 1.0716x; 1.0716x over previous
import functools

import jax
import jax.numpy as jnp
import numpy as np
from jax import lax
from jax.experimental import pallas as pl
from jax.experimental.pallas import tpu as pltpu
from jax.experimental.pallas import tpu_sc as plsc

N_NODES = 100000
EMBED_D = 64
BATCH = 1024
TILE_N = 1000
OFF_TILE = 2000
N_OFF_TILES = 15
N_OFF = OFF_TILE * N_OFF_TILES
COL_SPLIT = N_NODES - N_OFF
A_TILES = COL_SPLIT // TILE_N

_U32_9 = np.uint32(9)
_EXP_ONE = np.uint32(0x3F800000)
_MINVAL = np.float32(1e-10)
_SCALE = np.float32(np.float32(1.0) - np.float32(1e-10))
_LSE_SHIFT = np.float32(40.0)


def _threefry_bits(x1):
    ks = (np.uint32(0), np.uint32(1), np.uint32(0x1BD11BDB))
    rotations = ((13, 15, 26, 6), (17, 29, 16, 24))
    r = 13
    x0 = x1
    x1 = lax.shift_left(x1, np.uint32(r)) | lax.shift_right_logical(
        x1, np.uint32(32 - r))
    x1 = x1 ^ x0
    first = True
    for i in range(5):
        for r in rotations[i % 2]:
            if first:
                first = False
                continue
            x0 = x0 + x1
            x1 = lax.shift_left(x1, np.uint32(r)) | lax.shift_right_logical(
                x1, np.uint32(32 - r))
            x1 = x1 ^ x0
        x0 = x0 + ks[(i + 1) % 3]
        x1 = x1 + np.uint32(ks[(i + 2) % 3] + np.uint32(i + 1))
    return x0 ^ x1


def _bits_to_gumbel(bits):
    fbits = lax.shift_right_logical(bits, _U32_9) | _EXP_ONE
    u = lax.bitcast_convert_type(fbits, jnp.float32) - np.float32(1.0)
    u = jnp.maximum(_MINVAL, u * _SCALE + _MINVAL)
    return -jnp.log(-jnp.log(u))


def _online_update(j, tile_w, col0, scores, t, lane,
                   bv_ref, bi_ref, bs_ref, s_ref):
    tmax = jnp.max(t, axis=-1, keepdims=True)
    larg = jnp.min(jnp.where(t == tmax, lane, tile_w), axis=-1, keepdims=True)
    sel = jnp.sum(jnp.where(lane == larg, scores, 0.0), axis=-1, keepdims=True)
    upd = tmax > bv_ref[...]
    bv_ref[...] = jnp.where(upd, tmax, bv_ref[...])
    bi_ref[...] = jnp.where(upd, larg + (col0 + j * tile_w), bi_ref[...])
    bs_ref[...] = jnp.where(upd, sel, bs_ref[...])
    s_ref[...] += jnp.sum(jnp.exp(scores - _LSE_SHIFT), axis=-1, keepdims=True)


def _score_tile(ce_ref, emb_ref, bias_ref, tile_w):
    scores = lax.dot_general(
        ce_ref[...], emb_ref[...],
        dimension_numbers=(((1,), (1,)), ((), ())),
        preferred_element_type=jnp.float32)
    return scores + jnp.reshape(bias_ref[...], (1, tile_w))


def _a_body(ce_ref, emb_ref, bias_ref, bv_out, bi_out, bs_out, s_out,
            bv_ref, bi_ref, bs_ref, s_ref):
    j = pl.program_id(0)

    @pl.when(j == 0)
    def _init():
        bv_ref[...] = jnp.full((BATCH, 1), -jnp.inf, jnp.float32)
        bi_ref[...] = jnp.zeros((BATCH, 1), jnp.int32)
        bs_ref[...] = jnp.zeros((BATCH, 1), jnp.float32)
        s_ref[...] = jnp.zeros((BATCH, 1), jnp.float32)

    scores = _score_tile(ce_ref, emb_ref, bias_ref, TILE_N)
    lane = lax.broadcasted_iota(jnp.int32, (BATCH, TILE_N), 1)
    base = (lax.broadcasted_iota(jnp.int32, (BATCH, 1), 0) * N_NODES
            + (j * TILE_N + 1))
    bits = _threefry_bits((base + lane).astype(jnp.uint32))
    t = scores + _bits_to_gumbel(bits)
    _online_update(j, TILE_N, 0, scores, t, lane, bv_ref, bi_ref, bs_ref, s_ref)

    @pl.when(j == A_TILES - 1)
    def _finish():
        bv_out[...] = bv_ref[...]
        bi_out[...] = bi_ref[...]
        bs_out[...] = bs_ref[...]
        s_out[...] = s_ref[...]


def _b_body(ce_ref, emb_ref, bias_ref, bits_ref,
            bv_in, bi_in, bs_in, s_in, samp_ref, lp_ref,
            bv_ref, bi_ref, bs_ref, s_ref):
    j = pl.program_id(0)

    @pl.when(j == 0)
    def _init():
        bv_ref[...] = bv_in[...]
        bi_ref[...] = bi_in[...]
        bs_ref[...] = bs_in[...]
        s_ref[...] = s_in[...]

    scores = _score_tile(ce_ref, emb_ref, bias_ref, OFF_TILE)
    lane = lax.broadcasted_iota(jnp.int32, (BATCH, OFF_TILE), 1)
    t = scores + _bits_to_gumbel(jnp.reshape(bits_ref[...],
                                             (BATCH, OFF_TILE)))
    _online_update(j, OFF_TILE, COL_SPLIT, scores, t, lane,
                   bv_ref, bi_ref, bs_ref, s_ref)

    @pl.when(j == N_OFF_TILES - 1)
    def _finish():
        samp_ref[...] = bi_ref[...]
        lp_ref[...] = bs_ref[...] - (_LSE_SHIFT + jnp.log(s_ref[...]))


def _fused_tc(center_emb, embedding, bias, bits):
    state_shape = [jax.ShapeDtypeStruct((BATCH, 1), jnp.float32),
                   jax.ShapeDtypeStruct((BATCH, 1), jnp.int32),
                   jax.ShapeDtypeStruct((BATCH, 1), jnp.float32),
                   jax.ShapeDtypeStruct((BATCH, 1), jnp.float32)]
    state_specs = [pl.BlockSpec((BATCH, 1), lambda j: (0, 0))] * 4
    bias_a = bias[:COL_SPLIT].reshape(A_TILES, 1, TILE_N)
    bias_b = bias[COL_SPLIT:].reshape(N_OFF_TILES, 1, OFF_TILE)
    scratch = [pltpu.VMEM((BATCH, 1), jnp.float32),
               pltpu.VMEM((BATCH, 1), jnp.int32),
               pltpu.VMEM((BATCH, 1), jnp.float32),
               pltpu.VMEM((BATCH, 1), jnp.float32)]

    state = pl.pallas_call(
        _a_body,
        grid=(A_TILES,),
        in_specs=[
            pl.BlockSpec((BATCH, EMBED_D), lambda j: (0, 0)),
            pl.BlockSpec((TILE_N, EMBED_D), lambda j: (j, 0)),
            pl.BlockSpec((1, 1, TILE_N), lambda j: (j, 0, 0)),
        ],
        out_specs=state_specs,
        out_shape=state_shape,
        scratch_shapes=scratch,
        compiler_params=pltpu.CompilerParams(
            dimension_semantics=("arbitrary",)),
    )(center_emb, embedding, bias_a)

    samples2d, lp2d = pl.pallas_call(
        _b_body,
        grid=(N_OFF_TILES,),
        in_specs=[
            pl.BlockSpec((BATCH, EMBED_D), lambda j: (0, 0)),
            pl.BlockSpec((OFF_TILE, EMBED_D),
                         lambda j: (COL_SPLIT // OFF_TILE + j, 0)),
            pl.BlockSpec((1, 1, OFF_TILE), lambda j: (j, 0, 0)),
            pl.BlockSpec((1, BATCH, OFF_TILE), lambda j: (j, 0, 0)),
            *state_specs,
        ],
        out_specs=[
            pl.BlockSpec((BATCH, 1), lambda j: (0, 0)),
            pl.BlockSpec((BATCH, 1), lambda j: (0, 0)),
        ],
        out_shape=[
            jax.ShapeDtypeStruct((BATCH, 1), jnp.int32),
            jax.ShapeDtypeStruct((BATCH, 1), jnp.float32),
        ],
        scratch_shapes=scratch,
        compiler_params=pltpu.CompilerParams(
            dimension_semantics=("arbitrary",)),
    )(center_emb, embedding, bias_b, bits, *state)
    return samples2d[:, 0], lp2d[:, 0]


def _sc_bits():
    info = plsc.get_sparse_core_info()
    nc, ns = info.num_cores, info.num_subcores
    ntec = nc * ns
    rows_per_tec = BATCH // ntec
    mesh = plsc.VectorSubcoreMesh(core_axis_name="c", subcore_axis_name="s")

    @functools.partial(
        pl.kernel, mesh=mesh,
        out_type=jax.ShapeDtypeStruct((N_OFF_TILES, BATCH, OFF_TILE),
                                      jnp.uint32),
        scratch_types=[
            pltpu.VMEM((OFF_TILE,), jnp.uint32),
            pltpu.SemaphoreType.DMA,
        ],
    )
    def bits_kernel(out_hbm, buf, sem):
        wid = lax.axis_index("s") * nc + lax.axis_index("c")
        r0 = wid * rows_per_tec
        iota16 = lax.iota(jnp.int32, 16)
        unroll = 5

        def tile_body(rk, _):
            r = rk // N_OFF_TILES
            k = rk % N_OFF_TILES
            base = (r0 + r) * N_NODES + (COL_SPLIT + k * OFF_TILE + 1)

            def chunk(c, _):
                off = c * (16 * unroll)
                for v in range(unroll):
                    x1 = (base + (off + v * 16) + iota16).astype(jnp.uint32)
                    buf[pl.ds(off + v * 16, 16)] = _threefry_bits(x1)
                return ()

            lax.fori_loop(0, OFF_TILE // (16 * unroll), chunk, ())
            pltpu.async_copy(buf, out_hbm.at[k, r0 + r], sem).wait()
            return ()

        lax.fori_loop(0, rows_per_tec * N_OFF_TILES, tile_body, ())

    return bits_kernel()


def _gather_sc(center_ids, embedding):
    info = plsc.get_sparse_core_info()
    nc = info.num_cores
    b_per_c = BATCH // nc
    mesh = plsc.ScalarSubcoreMesh(axis_name="c", num_cores=nc)

    @functools.partial(
        pl.kernel, mesh=mesh,
        out_type=jax.ShapeDtypeStruct((BATCH, EMBED_D), jnp.float32),
        scratch_types=[
            pltpu.SMEM((b_per_c,), jnp.int32),
            pltpu.SemaphoreType.DMA,
            pltpu.SemaphoreType.DMA,
        ],
    )
    def gather(table_hbm, idx_hbm, out_hbm, idx_s, isem, sem):
        base = lax.axis_index("c") * b_per_c
        pltpu.async_copy(idx_hbm.at[pl.ds(base, b_per_c)], idx_s, isem).wait()

        def body(i, _):
            pltpu.async_copy(table_hbm.at[pl.ds(idx_s[i], 1), :],
                             out_hbm.at[pl.ds(base + i, 1), :], sem)
            return ()

        lax.fori_loop(0, b_per_c, body, (), unroll=8)
        pltpu.make_async_copy(
            table_hbm.at[pl.ds(0, b_per_c), :],
            out_hbm.at[pl.ds(base, b_per_c), :], sem).wait()

    return gather(embedding, center_ids)


def kernel(center_ids, embedding, bias):
    center_emb = _gather_sc(center_ids.astype(jnp.int32), embedding)
    bits = _sc_bits()
    return _fused_tc(center_emb, embedding, bias, bits)
